# baseline (speedup 1.0000x reference)
import functools

import jax
import jax.numpy as jnp
from jax import lax
from jax.experimental import pallas as pl
from jax.experimental.pallas import tpu as pltpu
from jax.experimental.pallas import tpu_sc as plsc

N_NODES = 10000
N_NET = 20000
N_EDGES = 320000
N_CONN = 80000
D = 128
H1 = 64

_NC, _NS = 2, 16
_NW = _NC * _NS
_EPW = N_EDGES // _NW
_CH = 80
_NCHUNK = _EPW // _CH
_RPT = 624
_TAIL0 = _NS * _RPT
_TAILN = N_NODES - _TAIL0


_HPAD = N_NODES + 16


def _seg_sum_body(x_hbm, src_hbm, dst_hbm, z128_hbm,
                  agg_out, deg_out0, deg_out1,
                  agg_sh, deg_st, src_v, dst_v, rows_v, hist_v, win_v, sem):
    cid = lax.axis_index("c")
    sid = lax.axis_index("s")
    wid = cid * _NS + sid
    pltpu.sync_copy(z128_hbm.at[pl.ds(0, _RPT)], agg_sh.at[pl.ds(sid * _RPT, _RPT)])

    @pl.when(sid == _NS - 1)
    def _zero_tail():
        pltpu.sync_copy(z128_hbm.at[pl.ds(0, _TAILN)], agg_sh.at[pl.ds(_TAIL0, _TAILN)])

    zeros16 = jnp.zeros((16,), jnp.float32)
    one0 = jnp.where(lax.iota(jnp.int32, 16) == 0, 1.0, 0.0).astype(jnp.float32)

    def zero_hist(i, c):
        hist_v[pl.ds(i * 16, 16)] = zeros16
        return c
    lax.fori_loop(0, _HPAD // 16, zero_hist, 0)
    plsc.subcore_barrier()

    base = wid * _EPW

    def step(i, c):
        off = base + i * _CH
        pltpu.sync_copy(src_hbm.at[pl.ds(off, _CH)], src_v)
        pltpu.sync_copy(dst_hbm.at[pl.ds(off, _CH)], dst_v)
        pltpu.async_copy(x_hbm.at[src_v], rows_v, sem).wait()
        pltpu.sync_copy(rows_v, agg_sh.at[dst_v], add=True)
        for g in range(_CH // 16):
            d_vec = dst_v[pl.ds(g * 16, 16)]
            for e in range(16):
                plsc.addupdate(hist_v.at[pl.ds(d_vec[e], 16)], one0)
        return c
    lax.fori_loop(0, _NCHUNK, step, 0)

    pltpu.sync_copy(hist_v.at[pl.ds(0, N_NODES)], deg_st.at[pl.ds(sid * N_NODES, N_NODES)])
    plsc.subcore_barrier()

    lo = sid * _RPT

    def reduce_window(lo, n, out_ref):
        pltpu.sync_copy(deg_st.at[pl.ds(lo, n)], win_v.at[pl.ds(0, n)])
        for j in range(1, _NS):
            pltpu.sync_copy(deg_st.at[pl.ds(j * N_NODES + lo, n)], win_v.at[pl.ds(n, n)])

            def acc(k, c):
                a = win_v[pl.ds(k * 16, 16)]
                b = win_v[pl.ds(n + k * 16, 16)]
                win_v[pl.ds(k * 16, 16)] = a + b
                return c
            lax.fori_loop(0, n // 16, acc, 0)
        pltpu.sync_copy(win_v.at[pl.ds(0, n)], out_ref.at[pl.ds(lo, n)])

    @pl.when(cid == 0)
    def _red0():
        reduce_window(lo, _RPT, deg_out0)

        @pl.when(sid == _NS - 1)
        def _tail0():
            reduce_window(_TAIL0, _TAILN, deg_out0)

    @pl.when(cid == 1)
    def _red1():
        reduce_window(lo, _RPT, deg_out1)

        @pl.when(sid == _NS - 1)
        def _tail1():
            reduce_window(_TAIL0, _TAILN, deg_out1)

    plsc.subcore_barrier()
    pltpu.sync_copy(agg_sh.at[pl.ds(sid * _RPT, _RPT)],
                    agg_out.at[cid, pl.ds(sid * _RPT, _RPT)])

    @pl.when(sid == _NS - 1)
    def _copy_tail():
        pltpu.sync_copy(agg_sh.at[pl.ds(_TAIL0, _TAILN)],
                        agg_out.at[cid, pl.ds(_TAIL0, _TAILN)])


def _seg_sum_sc(X, src, dst):
    mesh = plsc.VectorSubcoreMesh(core_axis_name="c", subcore_axis_name="s",
                                  num_cores=_NC, num_subcores=_NS)
    z128 = jnp.zeros((_RPT, D), jnp.float32)
    fn = pl.kernel(
        _seg_sum_body,
        out_type=[jax.ShapeDtypeStruct((_NC, N_NODES, D), jnp.float32),
                  jax.ShapeDtypeStruct((N_NODES,), jnp.float32),
                  jax.ShapeDtypeStruct((N_NODES,), jnp.float32)],
        mesh=mesh,
        scratch_types=[
            pltpu.VMEM_SHARED((N_NODES, D), jnp.float32),
            pltpu.VMEM_SHARED((_NS * N_NODES,), jnp.float32),
            pltpu.VMEM((_CH,), jnp.int32),
            pltpu.VMEM((_CH,), jnp.int32),
            pltpu.VMEM((_CH, D), jnp.float32),
            pltpu.VMEM((_HPAD,), jnp.float32),
            pltpu.VMEM((2 * _RPT,), jnp.float32),
            pltpu.SemaphoreType.DMA,
        ],
    )
    return fn(X, src, dst, z128)

_RB1 = 1000
_RB2 = 1000


_RPN = 624
_TAILC = N_NET - 31 * _RPN - _RPN
_OWN = _RPN + _TAILC
_ACCR = 664
_DUM = 656
_CB = 2000
_NCB = N_CONN // _CB
_GPB = _CB // 16
_GC = 128
_CAP = 2176
_CAPB = 2304


def _seg_max_body(h_hbm, csrc_hbm, cdst_hbm, ninf_hbm,
                  xx_out,
                  acc_v, srcb_v, dstb_v, cpk_v, rows_v, idx_v, sem):
    cid = lax.axis_index("c")
    sid = lax.axis_index("s")
    wid = cid * _NS + sid
    lo = wid * _RPN
    hi = lo + jnp.where(wid == _NW - 1, _OWN, _RPN)

    pltpu.sync_copy(ninf_hbm, acc_v)
    dumv = jnp.full((16,), _DUM, jnp.int32)

    def fill_dum(i, c):
        cpk_v[pl.ds(i * 16, 16)] = dumv
        return c
    lax.fori_loop(0, _CAPB // 16, fill_dum, 0)
    lov = jnp.zeros((16,), jnp.int32) + lo
    hiv = jnp.zeros((16,), jnp.int32) + hi

    def block(b, c):
        boff = b * _CB
        pltpu.sync_copy(csrc_hbm.at[pl.ds(boff, _CB)], srcb_v)
        pltpu.sync_copy(cdst_hbm.at[pl.ds(boff, _CB)], dstb_v)

        lanes = lax.iota(jnp.int32, 16)

        def compact(i, cnt):
            d = dstb_v[pl.ds(i * 16, 16)]
            s = srcb_v[pl.ds(i * 16, 16)]
            m = (d >= lo) & (d < hi)
            total = plsc.all_reduce_population_count(m)[0]

            @pl.when(total > 0)
            def _store():
                keys = jnp.where(m, lanes, lanes + 16)
                pkd = s * 1024 + (d - lo)
                _, spk = plsc.sort_key_val(keys, pkd)
                cpk_v[pl.ds(cnt, 16)] = spk
            return cnt + total
        cnt = lax.fori_loop(0, _GPB, compact, jnp.int32(0))

        for k in range(_GC // 16):
            cpk_v[pl.ds(cnt + k * 16, 16)] = dumv

        nchunks = (cnt + _GC - 1) // _GC

        for j in range(_CB // _GC + 1):
            @pl.when(j < nchunks)
            def _chunk():
                for g in range(_GC // 16):
                    idx_v[pl.ds(g * 16, 16)] = (
                        cpk_v[pl.ds(j * _GC + g * 16, 16)] >> 10)
                pltpu.async_copy(h_hbm.at[idx_v], rows_v, sem).wait()

                def group(g, c3):
                    dloc = cpk_v[pl.ds(j * _GC + g * 16, 16)] & 1023
                    for e in range(16):
                        dd = dloc[e]
                        r = g * 16 + e
                        for cc in range(8):
                            cs = pl.ds(cc * 16, 16)
                            acc_v[dd, cs] = jnp.maximum(acc_v[dd, cs],
                                                        rows_v[r, cs])
                    return c3
                lax.fori_loop(0, _GC // 16, group, 0)
        return c
    lax.fori_loop(0, _NCB, block, 0)

    pltpu.sync_copy(acc_v.at[pl.ds(0, _RPN)], xx_out.at[pl.ds(lo, _RPN)])

    @pl.when(wid == _NW - 1)
    def _tail():
        pltpu.sync_copy(acc_v.at[pl.ds(_RPN, _TAILC)],
                        xx_out.at[pl.ds(31 * _RPN + _RPN, _TAILC)])


def _seg_max_sc(h, conn_src, conn_dst):
    mesh = plsc.VectorSubcoreMesh(core_axis_name="c", subcore_axis_name="s",
                                  num_cores=_NC, num_subcores=_NS)
    ninf = jnp.full((_ACCR, D), -jnp.inf, jnp.float32)
    fn = pl.kernel(
        _seg_max_body,
        out_type=jax.ShapeDtypeStruct((N_NET, D), jnp.float32),
        mesh=mesh,
        scratch_types=[
            pltpu.VMEM((_ACCR, D), jnp.float32),
            pltpu.VMEM((_CB,), jnp.int32),
            pltpu.VMEM((_CB,), jnp.int32),
            pltpu.VMEM((_CAPB,), jnp.int32),
            pltpu.VMEM((_GC, D), jnp.float32),
            pltpu.VMEM((_GC,), jnp.int32),
            pltpu.SemaphoreType.DMA,
        ],
        compiler_params=pltpu.CompilerParams(needs_layout_passes=False),
    )
    return fn(h, conn_src, conn_dst, ninf)


def _sage_dense_body(x_ref, aggp_ref, degp_ref, ws_ref, wn_ref, b_ref, o_ref):
    agg = aggp_ref[0] + aggp_ref[1]
    deg = degp_ref[:, 0] + degp_ref[:, 1]
    inv = 1.0 / jnp.clip(deg, 1.0, None)
    hn = agg * inv[:, None]
    h = (jnp.dot(x_ref[...], ws_ref[...], preferred_element_type=jnp.float32)
         + jnp.dot(hn, wn_ref[...], preferred_element_type=jnp.float32)
         + b_ref[...])
    o_ref[...] = jnp.where(h >= 0.0, h, 0.01 * h)


def _sage_dense(X, agg_parts, deg_parts, W_self, W_neigh, b_sage):
    grid = (N_NODES // _RB1,)
    return pl.pallas_call(
        _sage_dense_body,
        grid=grid,
        in_specs=[
            pl.BlockSpec((_RB1, D), lambda i: (i, 0)),
            pl.BlockSpec((2, _RB1, D), lambda i: (0, i, 0)),
            pl.BlockSpec((_RB1, 2), lambda i: (i, 0)),
            pl.BlockSpec((D, D), lambda i: (0, 0)),
            pl.BlockSpec((D, D), lambda i: (0, 0)),
            pl.BlockSpec((1, D), lambda i: (0, 0)),
        ],
        out_specs=pl.BlockSpec((_RB1, D), lambda i: (i, 0)),
        out_shape=jax.ShapeDtypeStruct((N_NODES, D), jnp.float32),
    )(X, agg_parts, deg_parts, W_self, W_neigh, b_sage.reshape(1, D))


def _mlp_body(xx_ref, w1_ref, b1_ref, w2_ref, b2_ref, o_ref):
    xx = xx_ref[...]
    xx = jnp.where(xx == -jnp.inf, 0.0, xx)
    l1 = (jnp.dot(xx, w1_ref[...], preferred_element_type=jnp.float32)
          + b1_ref[...])
    l1 = jnp.where(l1 >= 0.0, l1, 0.01 * l1)
    o_ref[...] = jnp.tanh(
        jnp.dot(l1, w2_ref[...], preferred_element_type=jnp.float32) + b2_ref[...])


def _mlp(xx, W1, b1, W2, b2):
    grid = (N_NET // _RB2,)
    return pl.pallas_call(
        _mlp_body,
        grid=grid,
        in_specs=[
            pl.BlockSpec((_RB2, D), lambda i: (i, 0)),
            pl.BlockSpec((D, H1), lambda i: (0, 0)),
            pl.BlockSpec((1, H1), lambda i: (0, 0)),
            pl.BlockSpec((H1, 1), lambda i: (0, 0)),
            pl.BlockSpec((1, 1), lambda i: (0, 0)),
        ],
        out_specs=pl.BlockSpec((_RB2, 1), lambda i: (i, 0)),
        out_shape=jax.ShapeDtypeStruct((N_NET, 1), jnp.float32),
    )(xx, W1, b1.reshape(1, H1), W2, b2.reshape(1, 1))


def kernel(X, W_self, W_neigh, b_sage, W1, b1, W2, b2, edge_index, conn_src, conn_dst):
    src = edge_index[0]
    dst = edge_index[1]
    agg_parts, deg0, deg1 = _seg_sum_sc(X, src, dst)
    deg_parts = jnp.stack([deg0, deg1], axis=1)
    h = _sage_dense(X, agg_parts, deg_parts, W_self, W_neigh, b_sage)
    xx = _seg_max_sc(h, conn_src, conn_dst)
    return _mlp(xx, W1, b1, W2, b2)


# speedup vs baseline: 3.3019x; 3.3019x over previous
import functools

import jax
import jax.numpy as jnp
from jax import lax
from jax.experimental import pallas as pl
from jax.experimental.pallas import tpu as pltpu
from jax.experimental.pallas import tpu_sc as plsc

N_NODES = 10000
N_NET = 20000
N_EDGES = 320000
N_CONN = 80000
D = 128
H1 = 64

_NC, _NS = 2, 16
_NW = _NC * _NS
_EPW = N_EDGES // _NW
_CH = 80
_NCHUNK = _EPW // _CH
_RPT = 624
_TAIL0 = _NS * _RPT
_TAILN = N_NODES - _TAIL0


_HPAD = N_NODES + 16


def _seg_sum_body(x_hbm, src_hbm, dst_hbm, z128_hbm,
                  agg_out, deg_out0, deg_out1,
                  agg_sh, deg_st, src_v, dst_v, rows_v, hist_v, win_v, sem):
    cid = lax.axis_index("c")
    sid = lax.axis_index("s")
    wid = cid * _NS + sid
    pltpu.sync_copy(z128_hbm.at[pl.ds(0, _RPT)], agg_sh.at[pl.ds(sid * _RPT, _RPT)])

    @pl.when(sid == _NS - 1)
    def _zero_tail():
        pltpu.sync_copy(z128_hbm.at[pl.ds(0, _TAILN)], agg_sh.at[pl.ds(_TAIL0, _TAILN)])

    zeros16 = jnp.zeros((16,), jnp.float32)
    one0 = jnp.where(lax.iota(jnp.int32, 16) == 0, 1.0, 0.0).astype(jnp.float32)

    def zero_hist(i, c):
        hist_v[pl.ds(i * 16, 16)] = zeros16
        return c
    lax.fori_loop(0, _HPAD // 16, zero_hist, 0)
    plsc.subcore_barrier()

    base = wid * _EPW

    def step(i, c):
        off = base + i * _CH
        pltpu.sync_copy(src_hbm.at[pl.ds(off, _CH)], src_v)
        pltpu.sync_copy(dst_hbm.at[pl.ds(off, _CH)], dst_v)
        pltpu.async_copy(x_hbm.at[src_v], rows_v, sem).wait()
        pltpu.sync_copy(rows_v, agg_sh.at[dst_v], add=True)
        for g in range(_CH // 16):
            d_vec = dst_v[pl.ds(g * 16, 16)]
            for e in range(16):
                plsc.addupdate(hist_v.at[pl.ds(d_vec[e], 16)], one0)
        return c
    lax.fori_loop(0, _NCHUNK, step, 0)

    pltpu.sync_copy(hist_v.at[pl.ds(0, N_NODES)], deg_st.at[pl.ds(sid * N_NODES, N_NODES)])
    plsc.subcore_barrier()

    lo = sid * _RPT

    def reduce_window(lo, n, out_ref):
        pltpu.sync_copy(deg_st.at[pl.ds(lo, n)], win_v.at[pl.ds(0, n)])
        for j in range(1, _NS):
            pltpu.sync_copy(deg_st.at[pl.ds(j * N_NODES + lo, n)], win_v.at[pl.ds(n, n)])

            def acc(k, c):
                a = win_v[pl.ds(k * 16, 16)]
                b = win_v[pl.ds(n + k * 16, 16)]
                win_v[pl.ds(k * 16, 16)] = a + b
                return c
            lax.fori_loop(0, n // 16, acc, 0)
        pltpu.sync_copy(win_v.at[pl.ds(0, n)], out_ref.at[pl.ds(lo, n)])

    @pl.when(cid == 0)
    def _red0():
        reduce_window(lo, _RPT, deg_out0)

        @pl.when(sid == _NS - 1)
        def _tail0():
            reduce_window(_TAIL0, _TAILN, deg_out0)

    @pl.when(cid == 1)
    def _red1():
        reduce_window(lo, _RPT, deg_out1)

        @pl.when(sid == _NS - 1)
        def _tail1():
            reduce_window(_TAIL0, _TAILN, deg_out1)

    plsc.subcore_barrier()
    pltpu.sync_copy(agg_sh.at[pl.ds(sid * _RPT, _RPT)],
                    agg_out.at[cid, pl.ds(sid * _RPT, _RPT)])

    @pl.when(sid == _NS - 1)
    def _copy_tail():
        pltpu.sync_copy(agg_sh.at[pl.ds(_TAIL0, _TAILN)],
                        agg_out.at[cid, pl.ds(_TAIL0, _TAILN)])


def _seg_sum_sc(X, src, dst):
    mesh = plsc.VectorSubcoreMesh(core_axis_name="c", subcore_axis_name="s",
                                  num_cores=_NC, num_subcores=_NS)
    z128 = jnp.zeros((_RPT, D), jnp.float32)
    fn = pl.kernel(
        _seg_sum_body,
        out_type=[jax.ShapeDtypeStruct((_NC, N_NODES, D), jnp.float32),
                  jax.ShapeDtypeStruct((N_NODES,), jnp.float32),
                  jax.ShapeDtypeStruct((N_NODES,), jnp.float32)],
        mesh=mesh,
        scratch_types=[
            pltpu.VMEM_SHARED((N_NODES, D), jnp.float32),
            pltpu.VMEM_SHARED((_NS * N_NODES,), jnp.float32),
            pltpu.VMEM((_CH,), jnp.int32),
            pltpu.VMEM((_CH,), jnp.int32),
            pltpu.VMEM((_CH, D), jnp.float32),
            pltpu.VMEM((_HPAD,), jnp.float32),
            pltpu.VMEM((2 * _RPT,), jnp.float32),
            pltpu.SemaphoreType.DMA,
        ],
    )
    return fn(X, src, dst, z128)

_RB1 = 1000
_RB2 = 1000


_RPN = 624
_TAILC = N_NET - 31 * _RPN - _RPN
_OWN = _RPN + _TAILC
_ACCR = 664
_DUM = 656
_CB = 2000
_NCB = N_CONN // _CB
_GPB = _CB // 16
_GC = 128
_CAPL = 512
_DRTH = _CAPL - _GPB
_DUMPS = _CAPL * 16
_CAPB = 8320


def _seg_max_body(h_hbm, csrc_hbm, cdst_hbm, ninf_hbm,
                  xx_out,
                  acc_v, srcb_v, dstb_v, cpk_v, plc_v, rows_v, idx_v, sem):
    cid = lax.axis_index("c")
    sid = lax.axis_index("s")
    wid = cid * _NS + sid
    lo = wid * _RPN
    hi = lo + jnp.where(wid == _NW - 1, _OWN, _RPN)

    pltpu.sync_copy(ninf_hbm, acc_v)
    dumv = jnp.full((16,), _DUM, jnp.int32)

    def fill_dum(i, c):
        cpk_v[pl.ds(i * 16, 16)] = dumv
        return c
    lax.fori_loop(0, _CAPB // 16, fill_dum, 0)
    lanes = lax.iota(jnp.int32, 16)
    plc_v[...] = jnp.zeros((16,), jnp.int32)

    def block(b, c):
        @pl.when(b < _NCB)
        def _scan():
            boff = b * _CB
            pltpu.sync_copy(csrc_hbm.at[pl.ds(boff, _CB)], srcb_v)
            pltpu.sync_copy(cdst_hbm.at[pl.ds(boff, _CB)], dstb_v)

            def compact(i, c2):
                d = dstb_v[pl.ds(i * 16, 16)]
                sv = srcb_v[pl.ds(i * 16, 16)]
                m = (d >= lo) & (d < hi)
                plc = plc_v[...]
                slot = jnp.where(m, plc * 16 + lanes, _DUMPS)
                plsc.store_scatter(cpk_v, [slot], sv * 1024 + (d - lo))
                plc_v[...] = plc + jnp.where(m, 1, 0)
                return c2
            lax.fori_loop(0, _GPB, compact, 0)

        plc = plc_v[...]
        do_drain = jnp.any(plc >= _DRTH) | (b == _NCB)

        @pl.when(do_drain)
        def _drain():
            mx = plc[0]
            for e in range(1, 16):
                mx = jnp.maximum(mx, plc[e])
            nch = (mx + 7) // 8

            def chunk(rc, c4):
                @pl.when(rc < nch)
                def _chunk():
                    for g in range(_GC // 16):
                        idx_v[pl.ds(g * 16, 16)] = (
                            cpk_v[pl.ds(rc * _GC + g * 16, 16)] >> 10)
                    pltpu.async_copy(h_hbm.at[idx_v], rows_v, sem).wait()

                    def group(g, c3):
                        dloc = cpk_v[pl.ds(rc * _GC + g * 16, 16)] & 1023
                        for e in range(16):
                            dd = dloc[e]
                            r = g * 16 + e
                            for cc in range(8):
                                cs = pl.ds(cc * 16, 16)
                                acc_v[dd, cs] = jnp.maximum(acc_v[dd, cs],
                                                            rows_v[r, cs])
                        return c3
                    lax.fori_loop(0, _GC // 16, group, 0)
                return c4
            lax.fori_loop(0, _CAPL // 8, chunk, 0)
            plc_v[...] = jnp.zeros((16,), jnp.int32)
        return c
    lax.fori_loop(0, _NCB + 1, block, 0)

    pltpu.sync_copy(acc_v.at[pl.ds(0, _RPN)], xx_out.at[pl.ds(lo, _RPN)])

    @pl.when(wid == _NW - 1)
    def _tail():
        pltpu.sync_copy(acc_v.at[pl.ds(_RPN, _TAILC)],
                        xx_out.at[pl.ds(31 * _RPN + _RPN, _TAILC)])


def _seg_max_sc(h, conn_src, conn_dst):
    mesh = plsc.VectorSubcoreMesh(core_axis_name="c", subcore_axis_name="s",
                                  num_cores=_NC, num_subcores=_NS)
    ninf = jnp.full((_ACCR, D), -jnp.inf, jnp.float32)
    fn = pl.kernel(
        _seg_max_body,
        out_type=jax.ShapeDtypeStruct((N_NET, D), jnp.float32),
        mesh=mesh,
        scratch_types=[
            pltpu.VMEM((_ACCR, D), jnp.float32),
            pltpu.VMEM((_CB,), jnp.int32),
            pltpu.VMEM((_CB,), jnp.int32),
            pltpu.VMEM((_CAPB,), jnp.int32),
            pltpu.VMEM((16,), jnp.int32),
            pltpu.VMEM((_GC, D), jnp.float32),
            pltpu.VMEM((_GC,), jnp.int32),
            pltpu.SemaphoreType.DMA,
        ],
        compiler_params=pltpu.CompilerParams(needs_layout_passes=False),
    )
    return fn(h, conn_src, conn_dst, ninf)


def _sage_dense_body(x_ref, aggp_ref, degp_ref, ws_ref, wn_ref, b_ref, o_ref):
    agg = aggp_ref[0] + aggp_ref[1]
    deg = degp_ref[:, 0] + degp_ref[:, 1]
    inv = 1.0 / jnp.clip(deg, 1.0, None)
    hn = agg * inv[:, None]
    h = (jnp.dot(x_ref[...], ws_ref[...], preferred_element_type=jnp.float32)
         + jnp.dot(hn, wn_ref[...], preferred_element_type=jnp.float32)
         + b_ref[...])
    o_ref[...] = jnp.where(h >= 0.0, h, 0.01 * h)


def _sage_dense(X, agg_parts, deg_parts, W_self, W_neigh, b_sage):
    grid = (N_NODES // _RB1,)
    return pl.pallas_call(
        _sage_dense_body,
        grid=grid,
        in_specs=[
            pl.BlockSpec((_RB1, D), lambda i: (i, 0)),
            pl.BlockSpec((2, _RB1, D), lambda i: (0, i, 0)),
            pl.BlockSpec((_RB1, 2), lambda i: (i, 0)),
            pl.BlockSpec((D, D), lambda i: (0, 0)),
            pl.BlockSpec((D, D), lambda i: (0, 0)),
            pl.BlockSpec((1, D), lambda i: (0, 0)),
        ],
        out_specs=pl.BlockSpec((_RB1, D), lambda i: (i, 0)),
        out_shape=jax.ShapeDtypeStruct((N_NODES, D), jnp.float32),
    )(X, agg_parts, deg_parts, W_self, W_neigh, b_sage.reshape(1, D))


def _mlp_body(xx_ref, w1_ref, b1_ref, w2_ref, b2_ref, o_ref):
    xx = xx_ref[...]
    xx = jnp.where(xx == -jnp.inf, 0.0, xx)
    l1 = (jnp.dot(xx, w1_ref[...], preferred_element_type=jnp.float32)
          + b1_ref[...])
    l1 = jnp.where(l1 >= 0.0, l1, 0.01 * l1)
    o_ref[...] = jnp.tanh(
        jnp.dot(l1, w2_ref[...], preferred_element_type=jnp.float32) + b2_ref[...])


def _mlp(xx, W1, b1, W2, b2):
    grid = (N_NET // _RB2,)
    return pl.pallas_call(
        _mlp_body,
        grid=grid,
        in_specs=[
            pl.BlockSpec((_RB2, D), lambda i: (i, 0)),
            pl.BlockSpec((D, H1), lambda i: (0, 0)),
            pl.BlockSpec((1, H1), lambda i: (0, 0)),
            pl.BlockSpec((H1, 1), lambda i: (0, 0)),
            pl.BlockSpec((1, 1), lambda i: (0, 0)),
        ],
        out_specs=pl.BlockSpec((_RB2, 1), lambda i: (i, 0)),
        out_shape=jax.ShapeDtypeStruct((N_NET, 1), jnp.float32),
    )(xx, W1, b1.reshape(1, H1), W2, b2.reshape(1, 1))


def kernel(X, W_self, W_neigh, b_sage, W1, b1, W2, b2, edge_index, conn_src, conn_dst):
    src = edge_index[0]
    dst = edge_index[1]
    agg_parts, deg0, deg1 = _seg_sum_sc(X, src, dst)
    deg_parts = jnp.stack([deg0, deg1], axis=1)
    h = _sage_dense(X, agg_parts, deg_parts, W_self, W_neigh, b_sage)
    xx = _seg_max_sc(h, conn_src, conn_dst)
    return _mlp(xx, W1, b1, W2, b2)


# speedup vs baseline: 3.3959x; 1.0285x over previous
import functools

import jax
import jax.numpy as jnp
from jax import lax
from jax.experimental import pallas as pl
from jax.experimental.pallas import tpu as pltpu
from jax.experimental.pallas import tpu_sc as plsc

N_NODES = 10000
N_NET = 20000
N_EDGES = 320000
N_CONN = 80000
D = 128
H1 = 64

_NC, _NS = 2, 16
_NW = _NC * _NS
_EPW = N_EDGES // _NW
_CH = 80
_NCHUNK = _EPW // _CH
_RPT = 624
_TAIL0 = _NS * _RPT
_TAILN = N_NODES - _TAIL0


_HPAD = N_NODES + 16


def _seg_sum_body(x_hbm, src_hbm, dst_hbm, z128_hbm,
                  agg_out, deg_out0, deg_out1,
                  agg_sh, deg_st, src_v, dst_v, rows_v, hist_v, win_v, sem):
    cid = lax.axis_index("c")
    sid = lax.axis_index("s")
    wid = cid * _NS + sid
    pltpu.sync_copy(z128_hbm.at[pl.ds(0, _RPT)], agg_sh.at[pl.ds(sid * _RPT, _RPT)])

    @pl.when(sid == _NS - 1)
    def _zero_tail():
        pltpu.sync_copy(z128_hbm.at[pl.ds(0, _TAILN)], agg_sh.at[pl.ds(_TAIL0, _TAILN)])

    zeros16 = jnp.zeros((16,), jnp.float32)
    one0 = jnp.where(lax.iota(jnp.int32, 16) == 0, 1.0, 0.0).astype(jnp.float32)

    def zero_hist(i, c):
        hist_v[pl.ds(i * 16, 16)] = zeros16
        return c
    lax.fori_loop(0, _HPAD // 16, zero_hist, 0)
    plsc.subcore_barrier()

    base = wid * _EPW

    def step(i, c):
        off = base + i * _CH
        pltpu.sync_copy(src_hbm.at[pl.ds(off, _CH)], src_v)
        pltpu.sync_copy(dst_hbm.at[pl.ds(off, _CH)], dst_v)
        pltpu.async_copy(x_hbm.at[src_v], rows_v, sem).wait()
        pltpu.sync_copy(rows_v, agg_sh.at[dst_v], add=True)
        for g in range(_CH // 16):
            d_vec = dst_v[pl.ds(g * 16, 16)]
            for e in range(16):
                plsc.addupdate(hist_v.at[pl.ds(d_vec[e], 16)], one0)
        return c
    lax.fori_loop(0, _NCHUNK, step, 0)

    pltpu.sync_copy(hist_v.at[pl.ds(0, N_NODES)], deg_st.at[pl.ds(sid * N_NODES, N_NODES)])
    plsc.subcore_barrier()

    lo = sid * _RPT

    def reduce_window(lo, n, out_ref):
        pltpu.sync_copy(deg_st.at[pl.ds(lo, n)], win_v.at[pl.ds(0, n)])
        for j in range(1, _NS):
            pltpu.sync_copy(deg_st.at[pl.ds(j * N_NODES + lo, n)], win_v.at[pl.ds(n, n)])

            def acc(k, c):
                a = win_v[pl.ds(k * 16, 16)]
                b = win_v[pl.ds(n + k * 16, 16)]
                win_v[pl.ds(k * 16, 16)] = a + b
                return c
            lax.fori_loop(0, n // 16, acc, 0)
        pltpu.sync_copy(win_v.at[pl.ds(0, n)], out_ref.at[pl.ds(lo, n)])

    @pl.when(cid == 0)
    def _red0():
        reduce_window(lo, _RPT, deg_out0)

        @pl.when(sid == _NS - 1)
        def _tail0():
            reduce_window(_TAIL0, _TAILN, deg_out0)

    @pl.when(cid == 1)
    def _red1():
        reduce_window(lo, _RPT, deg_out1)

        @pl.when(sid == _NS - 1)
        def _tail1():
            reduce_window(_TAIL0, _TAILN, deg_out1)

    plsc.subcore_barrier()
    pltpu.sync_copy(agg_sh.at[pl.ds(sid * _RPT, _RPT)],
                    agg_out.at[cid, pl.ds(sid * _RPT, _RPT)])

    @pl.when(sid == _NS - 1)
    def _copy_tail():
        pltpu.sync_copy(agg_sh.at[pl.ds(_TAIL0, _TAILN)],
                        agg_out.at[cid, pl.ds(_TAIL0, _TAILN)])


def _seg_sum_sc(X, src, dst):
    mesh = plsc.VectorSubcoreMesh(core_axis_name="c", subcore_axis_name="s",
                                  num_cores=_NC, num_subcores=_NS)
    z128 = jnp.zeros((_RPT, D), jnp.float32)
    fn = pl.kernel(
        _seg_sum_body,
        out_type=[jax.ShapeDtypeStruct((_NC, N_NODES, D), jnp.float32),
                  jax.ShapeDtypeStruct((N_NODES,), jnp.float32),
                  jax.ShapeDtypeStruct((N_NODES,), jnp.float32)],
        mesh=mesh,
        scratch_types=[
            pltpu.VMEM_SHARED((N_NODES, D), jnp.float32),
            pltpu.VMEM_SHARED((_NS * N_NODES,), jnp.float32),
            pltpu.VMEM((_CH,), jnp.int32),
            pltpu.VMEM((_CH,), jnp.int32),
            pltpu.VMEM((_CH, D), jnp.float32),
            pltpu.VMEM((_HPAD,), jnp.float32),
            pltpu.VMEM((2 * _RPT,), jnp.float32),
            pltpu.SemaphoreType.DMA,
        ],
    )
    return fn(X, src, dst, z128)

_RB1 = 1000
_RB2 = 1000


_RPN = 624
_TAILC = N_NET - 31 * _RPN - _RPN
_OWN = _RPN + _TAILC
_ACCR = 664
_DUM = 656
_CB = 2000
_NCB = N_CONN // _CB
_GPB = _CB // 16
_GC = 128
_CAPL = 512
_DRTH = _CAPL - _GPB
_DUMPS = _CAPL * 16
_CAPB = 8320


def _seg_max_body(h_hbm, csrc_hbm, cdst_hbm, ninf_hbm,
                  xx_out,
                  conn_sh, acc_v, srcb_v, dstb_v, cpk_v, plc_v, rows_v, idx_v, sem):
    cid = lax.axis_index("c")
    sid = lax.axis_index("s")
    wid = cid * _NS + sid
    lo = wid * _RPN
    hi = lo + jnp.where(wid == _NW - 1, _OWN, _RPN)

    pltpu.sync_copy(ninf_hbm, acc_v)
    _cpt = N_CONN // _NS
    for hbm_ref, sbase in ((csrc_hbm, 0), (cdst_hbm, N_CONN)):
        for off, sz in ((0, 2000), (2000, 2000), (4000, 1000)):
            pltpu.sync_copy(hbm_ref.at[pl.ds(sid * _cpt + off, sz)],
                            srcb_v.at[pl.ds(0, sz)])
            pltpu.sync_copy(srcb_v.at[pl.ds(0, sz)],
                            conn_sh.at[pl.ds(sbase + sid * _cpt + off, sz)])
    dumv = jnp.full((16,), _DUM, jnp.int32)

    def fill_dum(i, c):
        cpk_v[pl.ds(i * 16, 16)] = dumv
        return c
    lax.fori_loop(0, _CAPB // 16, fill_dum, 0)
    lanes = lax.iota(jnp.int32, 16)
    plc_v[...] = jnp.zeros((16,), jnp.int32)
    plsc.subcore_barrier()

    def block(b, c):
        @pl.when(b < _NCB)
        def _scan():
            boff = b * _CB
            pltpu.sync_copy(conn_sh.at[pl.ds(boff, _CB)], srcb_v)
            pltpu.sync_copy(conn_sh.at[pl.ds(N_CONN + boff, _CB)], dstb_v)

            def compact(i, c2):
                d = dstb_v[pl.ds(i * 16, 16)]
                sv = srcb_v[pl.ds(i * 16, 16)]
                m = (d >= lo) & (d < hi)
                plc = plc_v[...]
                slot = jnp.where(m, plc * 16 + lanes, _DUMPS)
                plsc.store_scatter(cpk_v, [slot], sv * 1024 + (d - lo))
                plc_v[...] = plc + jnp.where(m, 1, 0)
                return c2
            lax.fori_loop(0, _GPB, compact, 0)

        plc = plc_v[...]
        do_drain = jnp.any(plc >= _DRTH) | (b == _NCB)

        @pl.when(do_drain)
        def _drain():
            mx = plc[0]
            for e in range(1, 16):
                mx = jnp.maximum(mx, plc[e])
            nch = (mx + 7) // 8

            def chunk(rc, c4):
                @pl.when(rc < nch)
                def _chunk():
                    for g in range(_GC // 16):
                        idx_v[pl.ds(g * 16, 16)] = (
                            cpk_v[pl.ds(rc * _GC + g * 16, 16)] >> 10)
                    pltpu.async_copy(h_hbm.at[idx_v], rows_v, sem).wait()

                    def group(g, c3):
                        dloc = cpk_v[pl.ds(rc * _GC + g * 16, 16)] & 1023
                        for e in range(16):
                            dd = dloc[e]
                            r = g * 16 + e
                            for cc in range(8):
                                cs = pl.ds(cc * 16, 16)
                                acc_v[dd, cs] = jnp.maximum(acc_v[dd, cs],
                                                            rows_v[r, cs])
                        return c3
                    lax.fori_loop(0, _GC // 16, group, 0)
                return c4
            lax.fori_loop(0, _CAPL // 8, chunk, 0)
            plc_v[...] = jnp.zeros((16,), jnp.int32)
        return c
    lax.fori_loop(0, _NCB + 1, block, 0)

    pltpu.sync_copy(acc_v.at[pl.ds(0, _RPN)], xx_out.at[pl.ds(lo, _RPN)])

    @pl.when(wid == _NW - 1)
    def _tail():
        pltpu.sync_copy(acc_v.at[pl.ds(_RPN, _TAILC)],
                        xx_out.at[pl.ds(31 * _RPN + _RPN, _TAILC)])


def _seg_max_sc(h, conn_src, conn_dst):
    mesh = plsc.VectorSubcoreMesh(core_axis_name="c", subcore_axis_name="s",
                                  num_cores=_NC, num_subcores=_NS)
    ninf = jnp.full((_ACCR, D), -jnp.inf, jnp.float32)
    fn = pl.kernel(
        _seg_max_body,
        out_type=jax.ShapeDtypeStruct((N_NET, D), jnp.float32),
        mesh=mesh,
        scratch_types=[
            pltpu.VMEM_SHARED((2 * N_CONN,), jnp.int32),
            pltpu.VMEM((_ACCR, D), jnp.float32),
            pltpu.VMEM((_CB,), jnp.int32),
            pltpu.VMEM((_CB,), jnp.int32),
            pltpu.VMEM((_CAPB,), jnp.int32),
            pltpu.VMEM((16,), jnp.int32),
            pltpu.VMEM((_GC, D), jnp.float32),
            pltpu.VMEM((_GC,), jnp.int32),
            pltpu.SemaphoreType.DMA,
        ],
        compiler_params=pltpu.CompilerParams(needs_layout_passes=False),
    )
    return fn(h, conn_src, conn_dst, ninf)


def _sage_dense_body(x_ref, aggp_ref, degp_ref, ws_ref, wn_ref, b_ref, o_ref):
    agg = aggp_ref[0] + aggp_ref[1]
    deg = degp_ref[:, 0] + degp_ref[:, 1]
    inv = 1.0 / jnp.clip(deg, 1.0, None)
    hn = agg * inv[:, None]
    h = (jnp.dot(x_ref[...], ws_ref[...], preferred_element_type=jnp.float32)
         + jnp.dot(hn, wn_ref[...], preferred_element_type=jnp.float32)
         + b_ref[...])
    o_ref[...] = jnp.where(h >= 0.0, h, 0.01 * h)


def _sage_dense(X, agg_parts, deg_parts, W_self, W_neigh, b_sage):
    grid = (N_NODES // _RB1,)
    return pl.pallas_call(
        _sage_dense_body,
        grid=grid,
        in_specs=[
            pl.BlockSpec((_RB1, D), lambda i: (i, 0)),
            pl.BlockSpec((2, _RB1, D), lambda i: (0, i, 0)),
            pl.BlockSpec((_RB1, 2), lambda i: (i, 0)),
            pl.BlockSpec((D, D), lambda i: (0, 0)),
            pl.BlockSpec((D, D), lambda i: (0, 0)),
            pl.BlockSpec((1, D), lambda i: (0, 0)),
        ],
        out_specs=pl.BlockSpec((_RB1, D), lambda i: (i, 0)),
        out_shape=jax.ShapeDtypeStruct((N_NODES, D), jnp.float32),
    )(X, agg_parts, deg_parts, W_self, W_neigh, b_sage.reshape(1, D))


def _mlp_body(xx_ref, w1_ref, b1_ref, w2_ref, b2_ref, o_ref):
    xx = xx_ref[...]
    xx = jnp.where(xx == -jnp.inf, 0.0, xx)
    l1 = (jnp.dot(xx, w1_ref[...], preferred_element_type=jnp.float32)
          + b1_ref[...])
    l1 = jnp.where(l1 >= 0.0, l1, 0.01 * l1)
    o_ref[...] = jnp.tanh(
        jnp.dot(l1, w2_ref[...], preferred_element_type=jnp.float32) + b2_ref[...])


def _mlp(xx, W1, b1, W2, b2):
    grid = (N_NET // _RB2,)
    return pl.pallas_call(
        _mlp_body,
        grid=grid,
        in_specs=[
            pl.BlockSpec((_RB2, D), lambda i: (i, 0)),
            pl.BlockSpec((D, H1), lambda i: (0, 0)),
            pl.BlockSpec((1, H1), lambda i: (0, 0)),
            pl.BlockSpec((H1, 1), lambda i: (0, 0)),
            pl.BlockSpec((1, 1), lambda i: (0, 0)),
        ],
        out_specs=pl.BlockSpec((_RB2, 1), lambda i: (i, 0)),
        out_shape=jax.ShapeDtypeStruct((N_NET, 1), jnp.float32),
    )(xx, W1, b1.reshape(1, H1), W2, b2.reshape(1, 1))


def kernel(X, W_self, W_neigh, b_sage, W1, b1, W2, b2, edge_index, conn_src, conn_dst):
    src = edge_index[0]
    dst = edge_index[1]
    agg_parts, deg0, deg1 = _seg_sum_sc(X, src, dst)
    deg_parts = jnp.stack([deg0, deg1], axis=1)
    h = _sage_dense(X, agg_parts, deg_parts, W_self, W_neigh, b_sage)
    xx = _seg_max_sc(h, conn_src, conn_dst)
    return _mlp(xx, W1, b1, W2, b2)


# speedup vs baseline: 4.2122x; 1.2404x over previous
import functools

import jax
import jax.numpy as jnp
from jax import lax
from jax.experimental import pallas as pl
from jax.experimental.pallas import tpu as pltpu
from jax.experimental.pallas import tpu_sc as plsc

N_NODES = 10000
N_NET = 20000
N_EDGES = 320000
N_CONN = 80000
D = 128
H1 = 64

_NC, _NS = 2, 16
_NW = _NC * _NS
_EPW = N_EDGES // _NW
_CH = 80
_NCHUNK = _EPW // _CH
_RPT = 624
_TAIL0 = _NS * _RPT
_TAILN = N_NODES - _TAIL0


_DT = 624


def _seg_sum_body(x_hbm, src_hbm, dst_hbm, z128_hbm,
                  agg_out, deg_out0, deg_out1,
                  agg_sh, deg_sh, src_all, dst_all,
                  gidx0, gidx1, didx0, didx1, rows0, rows1,
                  ones_v, dtmp_v, gs0, gs1, ss0, ss1, os0, os1):
    cid = lax.axis_index("c")
    sid = lax.axis_index("s")
    wid = cid * _NS + sid
    pltpu.sync_copy(z128_hbm.at[pl.ds(0, _RPT)], agg_sh.at[pl.ds(sid * _RPT, _RPT)])

    @pl.when(sid == _NS - 1)
    def _zero_tail():
        pltpu.sync_copy(z128_hbm.at[pl.ds(0, _TAILN)], agg_sh.at[pl.ds(_TAIL0, _TAILN)])

    zeros16 = jnp.zeros((16,), jnp.float32)
    for g in range(_DT // 16):
        dtmp_v[pl.ds(g * 16, 16)] = zeros16
    for g in range(_CH // 16):
        ones_v[pl.ds(g * 16, 16)] = jnp.full((16,), 1.0, jnp.float32)
    pltpu.sync_copy(dtmp_v.at[pl.ds(0, _RPT)], deg_sh.at[pl.ds(sid * _RPT, _RPT)])

    @pl.when(sid == _NS - 1)
    def _zero_dtail():
        pltpu.sync_copy(dtmp_v.at[pl.ds(0, _TAILN)], deg_sh.at[pl.ds(_TAIL0, _TAILN)])

    base = wid * _EPW
    pltpu.sync_copy(src_hbm.at[pl.ds(base, _EPW)], src_all)
    pltpu.sync_copy(dst_hbm.at[pl.ds(base, _EPW)], dst_all)
    plsc.subcore_barrier()

    gidx = (gidx0, gidx1)
    didx = (didx0, didx1)
    rows = (rows0, rows1)
    gsem = (gs0, gs1)
    ssem = (ss0, ss1)
    osem = (os0, os1)

    def stage_gidx(k, p):
        for g in range(_CH // 16):
            gidx[p][pl.ds(g * 16, 16)] = src_all[pl.ds(k * _CH + g * 16, 16)]

    def gather(k, p):
        return pltpu.async_copy(x_hbm.at[gidx[p]], rows[p], gsem[p])

    stage_gidx(0, 0)
    gather(0, 0)
    stage_gidx(1, 1)
    gather(1, 1)

    def pair(i, c):
        for p in range(2):
            k = 2 * i + p

            @pl.when(k < _NCHUNK)
            def _sub():
                pltpu.make_async_copy(x_hbm.at[gidx[p]], rows[p], gsem[p]).wait()
                for g in range(_CH // 16):
                    didx[p][pl.ds(g * 16, 16)] = dst_all[pl.ds(k * _CH + g * 16, 16)]
                sdesc = pltpu.async_copy(rows[p], agg_sh.at[didx[p]], ssem[p],
                                         add=True)
                odesc = pltpu.async_copy(ones_v, deg_sh.at[didx[p]], osem[p],
                                         add=True)
                sdesc.wait()
                odesc.wait()

                @pl.when(k + 2 < _NCHUNK)
                def _next():
                    stage_gidx(k + 2, p)
                    gather(k + 2, p)
        return c
    lax.fori_loop(0, (_NCHUNK + 1) // 2, pair, 0)
    plsc.subcore_barrier()

    pltpu.sync_copy(agg_sh.at[pl.ds(sid * _RPT, _RPT)],
                    agg_out.at[cid, pl.ds(sid * _RPT, _RPT)])
    pltpu.sync_copy(deg_sh.at[pl.ds(sid * _RPT, _RPT)], dtmp_v.at[pl.ds(0, _RPT)])

    @pl.when(cid == 0)
    def _dout0():
        pltpu.sync_copy(dtmp_v.at[pl.ds(0, _RPT)], deg_out0.at[pl.ds(sid * _RPT, _RPT)])

    @pl.when(cid == 1)
    def _dout1():
        pltpu.sync_copy(dtmp_v.at[pl.ds(0, _RPT)], deg_out1.at[pl.ds(sid * _RPT, _RPT)])

    @pl.when(sid == _NS - 1)
    def _copy_tail():
        pltpu.sync_copy(agg_sh.at[pl.ds(_TAIL0, _TAILN)],
                        agg_out.at[cid, pl.ds(_TAIL0, _TAILN)])
        pltpu.sync_copy(deg_sh.at[pl.ds(_TAIL0, _TAILN)], dtmp_v.at[pl.ds(0, _TAILN)])

        @pl.when(cid == 0)
        def _dt0():
            pltpu.sync_copy(dtmp_v.at[pl.ds(0, _TAILN)], deg_out0.at[pl.ds(_TAIL0, _TAILN)])

        @pl.when(cid == 1)
        def _dt1():
            pltpu.sync_copy(dtmp_v.at[pl.ds(0, _TAILN)], deg_out1.at[pl.ds(_TAIL0, _TAILN)])


def _seg_sum_sc(X, src, dst):
    mesh = plsc.VectorSubcoreMesh(core_axis_name="c", subcore_axis_name="s",
                                  num_cores=_NC, num_subcores=_NS)
    z128 = jnp.zeros((_RPT, D), jnp.float32)
    fn = pl.kernel(
        _seg_sum_body,
        out_type=[jax.ShapeDtypeStruct((_NC, N_NODES, D), jnp.float32),
                  jax.ShapeDtypeStruct((N_NODES,), jnp.float32),
                  jax.ShapeDtypeStruct((N_NODES,), jnp.float32)],
        mesh=mesh,
        scratch_types=[
            pltpu.VMEM_SHARED((N_NODES, D), jnp.float32),
            pltpu.VMEM_SHARED((N_NODES,), jnp.float32),
            pltpu.VMEM((_EPW,), jnp.int32),
            pltpu.VMEM((_EPW,), jnp.int32),
            pltpu.VMEM((_CH,), jnp.int32),
            pltpu.VMEM((_CH,), jnp.int32),
            pltpu.VMEM((_CH,), jnp.int32),
            pltpu.VMEM((_CH,), jnp.int32),
            pltpu.VMEM((_CH, D), jnp.float32),
            pltpu.VMEM((_CH, D), jnp.float32),
            pltpu.VMEM((_CH,), jnp.float32),
            pltpu.VMEM((_DT,), jnp.float32),
            pltpu.SemaphoreType.DMA,
            pltpu.SemaphoreType.DMA,
            pltpu.SemaphoreType.DMA,
            pltpu.SemaphoreType.DMA,
            pltpu.SemaphoreType.DMA,
            pltpu.SemaphoreType.DMA,
        ],
    )
    return fn(X, src, dst, z128)

_RB1 = 1000
_RB2 = 1000


_RPN = 624
_TAILC = N_NET - 31 * _RPN - _RPN
_OWN = _RPN + _TAILC
_ACCR = 664
_DUM = 656
_CB = 2000
_NCB = N_CONN // _CB
_GPB = _CB // 16
_GC = 128
_CAPL = 384
_DRTH = _CAPL - _GPB
_DUMPS = _CAPL * 16
_CAPB = 6272


def _seg_max_body(h_hbm, csrc_hbm, cdst_hbm, ninf_hbm,
                  xx_out,
                  conn_sh, acc_v, srcb_v, dstb_v, cpk_v, plc_v, rows_v, idx_v, sem):
    cid = lax.axis_index("c")
    sid = lax.axis_index("s")
    wid = cid * _NS + sid
    lo = wid * _RPN
    hi = lo + jnp.where(wid == _NW - 1, _OWN, _RPN)

    pltpu.sync_copy(ninf_hbm, acc_v)
    _cpt = N_CONN // _NS
    for hbm_ref, sbase in ((csrc_hbm, 0), (cdst_hbm, N_CONN)):
        for off, sz in ((0, 2000), (2000, 2000), (4000, 1000)):
            pltpu.sync_copy(hbm_ref.at[pl.ds(sid * _cpt + off, sz)],
                            srcb_v.at[pl.ds(0, sz)])
            pltpu.sync_copy(srcb_v.at[pl.ds(0, sz)],
                            conn_sh.at[pl.ds(sbase + sid * _cpt + off, sz)])
    dumv = jnp.full((16,), _DUM, jnp.int32)

    def fill_dum(i, c):
        cpk_v[pl.ds(i * 16, 16)] = dumv
        return c
    lax.fori_loop(0, _CAPB // 16, fill_dum, 0)
    lanes = lax.iota(jnp.int32, 16)
    plc_v[...] = jnp.zeros((16,), jnp.int32)
    plsc.subcore_barrier()

    def block(b, c):
        @pl.when(b < _NCB)
        def _scan():
            boff = b * _CB
            pltpu.sync_copy(conn_sh.at[pl.ds(boff, _CB)], srcb_v)
            pltpu.sync_copy(conn_sh.at[pl.ds(N_CONN + boff, _CB)], dstb_v)

            def compact(i, c2):
                d = dstb_v[pl.ds(i * 16, 16)]
                sv = srcb_v[pl.ds(i * 16, 16)]
                m = (d >= lo) & (d < hi)
                plc = plc_v[...]
                slot = jnp.where(m, plc * 16 + lanes, _DUMPS)
                plsc.store_scatter(cpk_v, [slot], sv * 1024 + (d - lo))
                plc_v[...] = plc + jnp.where(m, 1, 0)
                return c2
            lax.fori_loop(0, _GPB, compact, 0)

        plc = plc_v[...]
        do_drain = jnp.any(plc >= _DRTH) | (b == _NCB)

        @pl.when(do_drain)
        def _drain():
            mx = plc[0]
            for e in range(1, 16):
                mx = jnp.maximum(mx, plc[e])
            nch = (mx + 7) // 8

            def chunk(rc, c4):
                @pl.when(rc < nch)
                def _chunk():
                    for g in range(_GC // 16):
                        idx_v[pl.ds(g * 16, 16)] = (
                            cpk_v[pl.ds(rc * _GC + g * 16, 16)] >> 10)
                    pltpu.async_copy(h_hbm.at[idx_v], rows_v, sem).wait()

                    def group(g, c3):
                        dloc = cpk_v[pl.ds(rc * _GC + g * 16, 16)] & 1023
                        for e in range(16):
                            dd = dloc[e]
                            r = g * 16 + e
                            for cc in range(8):
                                cs = pl.ds(cc * 16, 16)
                                acc_v[dd, cs] = jnp.maximum(acc_v[dd, cs],
                                                            rows_v[r, cs])
                        return c3
                    lax.fori_loop(0, _GC // 16, group, 0)
                return c4
            lax.fori_loop(0, _CAPL // 8, chunk, 0)
            plc_v[...] = jnp.zeros((16,), jnp.int32)
        return c
    lax.fori_loop(0, _NCB + 1, block, 0)

    pltpu.sync_copy(acc_v.at[pl.ds(0, _RPN)], xx_out.at[pl.ds(lo, _RPN)])

    @pl.when(wid == _NW - 1)
    def _tail():
        pltpu.sync_copy(acc_v.at[pl.ds(_RPN, _TAILC)],
                        xx_out.at[pl.ds(31 * _RPN + _RPN, _TAILC)])


def _seg_max_sc(h, conn_src, conn_dst):
    mesh = plsc.VectorSubcoreMesh(core_axis_name="c", subcore_axis_name="s",
                                  num_cores=_NC, num_subcores=_NS)
    ninf = jnp.full((_ACCR, D), -jnp.inf, jnp.float32)
    fn = pl.kernel(
        _seg_max_body,
        out_type=jax.ShapeDtypeStruct((N_NET, D), jnp.float32),
        mesh=mesh,
        scratch_types=[
            pltpu.VMEM_SHARED((2 * N_CONN,), jnp.int32),
            pltpu.VMEM((_ACCR, D), jnp.float32),
            pltpu.VMEM((_CB,), jnp.int32),
            pltpu.VMEM((_CB,), jnp.int32),
            pltpu.VMEM((_CAPB,), jnp.int32),
            pltpu.VMEM((16,), jnp.int32),
            pltpu.VMEM((_GC, D), jnp.float32),
            pltpu.VMEM((_GC,), jnp.int32),
            pltpu.SemaphoreType.DMA,
        ],
        compiler_params=pltpu.CompilerParams(needs_layout_passes=False),
    )
    return fn(h, conn_src, conn_dst, ninf)


def _sage_dense_body(x_ref, aggp_ref, degp_ref, ws_ref, wn_ref, b_ref, o_ref):
    agg = aggp_ref[0] + aggp_ref[1]
    deg = degp_ref[:, 0] + degp_ref[:, 1]
    inv = 1.0 / jnp.clip(deg, 1.0, None)
    hn = agg * inv[:, None]
    h = (jnp.dot(x_ref[...], ws_ref[...], preferred_element_type=jnp.float32)
         + jnp.dot(hn, wn_ref[...], preferred_element_type=jnp.float32)
         + b_ref[...])
    o_ref[...] = jnp.where(h >= 0.0, h, 0.01 * h)


def _sage_dense(X, agg_parts, deg_parts, W_self, W_neigh, b_sage):
    grid = (N_NODES // _RB1,)
    return pl.pallas_call(
        _sage_dense_body,
        grid=grid,
        in_specs=[
            pl.BlockSpec((_RB1, D), lambda i: (i, 0)),
            pl.BlockSpec((2, _RB1, D), lambda i: (0, i, 0)),
            pl.BlockSpec((_RB1, 2), lambda i: (i, 0)),
            pl.BlockSpec((D, D), lambda i: (0, 0)),
            pl.BlockSpec((D, D), lambda i: (0, 0)),
            pl.BlockSpec((1, D), lambda i: (0, 0)),
        ],
        out_specs=pl.BlockSpec((_RB1, D), lambda i: (i, 0)),
        out_shape=jax.ShapeDtypeStruct((N_NODES, D), jnp.float32),
    )(X, agg_parts, deg_parts, W_self, W_neigh, b_sage.reshape(1, D))


def _mlp_body(xx_ref, w1_ref, b1_ref, w2_ref, b2_ref, o_ref):
    xx = xx_ref[...]
    xx = jnp.where(xx == -jnp.inf, 0.0, xx)
    l1 = (jnp.dot(xx, w1_ref[...], preferred_element_type=jnp.float32)
          + b1_ref[...])
    l1 = jnp.where(l1 >= 0.0, l1, 0.01 * l1)
    o_ref[...] = jnp.tanh(
        jnp.dot(l1, w2_ref[...], preferred_element_type=jnp.float32) + b2_ref[...])


def _mlp(xx, W1, b1, W2, b2):
    grid = (N_NET // _RB2,)
    return pl.pallas_call(
        _mlp_body,
        grid=grid,
        in_specs=[
            pl.BlockSpec((_RB2, D), lambda i: (i, 0)),
            pl.BlockSpec((D, H1), lambda i: (0, 0)),
            pl.BlockSpec((1, H1), lambda i: (0, 0)),
            pl.BlockSpec((H1, 1), lambda i: (0, 0)),
            pl.BlockSpec((1, 1), lambda i: (0, 0)),
        ],
        out_specs=pl.BlockSpec((_RB2, 1), lambda i: (i, 0)),
        out_shape=jax.ShapeDtypeStruct((N_NET, 1), jnp.float32),
    )(xx, W1, b1.reshape(1, H1), W2, b2.reshape(1, 1))


def kernel(X, W_self, W_neigh, b_sage, W1, b1, W2, b2, edge_index, conn_src, conn_dst):
    src = edge_index[0]
    dst = edge_index[1]
    agg_parts, deg0, deg1 = _seg_sum_sc(X, src, dst)
    deg_parts = jnp.stack([deg0, deg1], axis=1)
    h = _sage_dense(X, agg_parts, deg_parts, W_self, W_neigh, b_sage)
    xx = _seg_max_sc(h, conn_src, conn_dst)
    return _mlp(xx, W1, b1, W2, b2)


# speedup vs baseline: 4.2721x; 1.0142x over previous
import functools

import jax
import jax.numpy as jnp
from jax import lax
from jax.experimental import pallas as pl
from jax.experimental.pallas import tpu as pltpu
from jax.experimental.pallas import tpu_sc as plsc

N_NODES = 10000
N_NET = 20000
N_EDGES = 320000
N_CONN = 80000
D = 128
H1 = 64

_NC, _NS = 2, 16
_NW = _NC * _NS
_EPW = N_EDGES // _NW
_CH = 80
_NCHUNK = _EPW // _CH
_RPT = 624
_TAIL0 = _NS * _RPT
_TAILN = N_NODES - _TAIL0


_DT = 624


def _seg_sum_body(x_hbm, src_hbm, dst_hbm, z128_hbm,
                  agg_out, deg_out0, deg_out1,
                  agg_sh, deg_sh, src_all, dst_all,
                  gidx0, gidx1, didx0, didx1, rows0, rows1,
                  ones_v, dtmp_v, gs0, gs1, ss0, ss1, os0, os1):
    cid = lax.axis_index("c")
    sid = lax.axis_index("s")
    wid = cid * _NS + sid
    pltpu.sync_copy(z128_hbm.at[pl.ds(0, _RPT)], agg_sh.at[pl.ds(sid * _RPT, _RPT)])

    @pl.when(sid == _NS - 1)
    def _zero_tail():
        pltpu.sync_copy(z128_hbm.at[pl.ds(0, _TAILN)], agg_sh.at[pl.ds(_TAIL0, _TAILN)])

    zeros16 = jnp.zeros((16,), jnp.float32)
    for g in range(_DT // 16):
        dtmp_v[pl.ds(g * 16, 16)] = zeros16
    for g in range(_CH // 16):
        ones_v[pl.ds(g * 16, 16)] = jnp.full((16,), 1.0, jnp.float32)
    pltpu.sync_copy(dtmp_v.at[pl.ds(0, _RPT)], deg_sh.at[pl.ds(sid * _RPT, _RPT)])

    @pl.when(sid == _NS - 1)
    def _zero_dtail():
        pltpu.sync_copy(dtmp_v.at[pl.ds(0, _TAILN)], deg_sh.at[pl.ds(_TAIL0, _TAILN)])

    base = wid * _EPW
    pltpu.sync_copy(src_hbm.at[pl.ds(base, _EPW)], src_all)
    pltpu.sync_copy(dst_hbm.at[pl.ds(base, _EPW)], dst_all)
    plsc.subcore_barrier()

    gidx = (gidx0, gidx1)
    didx = (didx0, didx1)
    rows = (rows0, rows1)
    gsem = (gs0, gs1)
    ssem = (ss0, ss1)
    osem = (os0, os1)

    def stage_gidx(k, p):
        for g in range(_CH // 16):
            gidx[p][pl.ds(g * 16, 16)] = src_all[pl.ds(k * _CH + g * 16, 16)]

    def gather(k, p):
        return pltpu.async_copy(x_hbm.at[gidx[p]], rows[p], gsem[p])

    stage_gidx(0, 0)
    gather(0, 0)
    stage_gidx(1, 1)
    gather(1, 1)

    def pair(i, c):
        for p in range(2):
            k = 2 * i + p

            @pl.when(k < _NCHUNK)
            def _sub():
                pltpu.make_async_copy(x_hbm.at[gidx[p]], rows[p], gsem[p]).wait()
                for g in range(_CH // 16):
                    didx[p][pl.ds(g * 16, 16)] = dst_all[pl.ds(k * _CH + g * 16, 16)]
                sdesc = pltpu.async_copy(rows[p], agg_sh.at[didx[p]], ssem[p],
                                         add=True)
                odesc = pltpu.async_copy(ones_v, deg_sh.at[didx[p]], osem[p],
                                         add=True)
                sdesc.wait()
                odesc.wait()

                @pl.when(k + 2 < _NCHUNK)
                def _next():
                    stage_gidx(k + 2, p)
                    gather(k + 2, p)
        return c
    lax.fori_loop(0, (_NCHUNK + 1) // 2, pair, 0)
    plsc.subcore_barrier()

    pltpu.sync_copy(agg_sh.at[pl.ds(sid * _RPT, _RPT)],
                    agg_out.at[cid, pl.ds(sid * _RPT, _RPT)])
    pltpu.sync_copy(deg_sh.at[pl.ds(sid * _RPT, _RPT)], dtmp_v.at[pl.ds(0, _RPT)])

    @pl.when(cid == 0)
    def _dout0():
        pltpu.sync_copy(dtmp_v.at[pl.ds(0, _RPT)], deg_out0.at[pl.ds(sid * _RPT, _RPT)])

    @pl.when(cid == 1)
    def _dout1():
        pltpu.sync_copy(dtmp_v.at[pl.ds(0, _RPT)], deg_out1.at[pl.ds(sid * _RPT, _RPT)])

    @pl.when(sid == _NS - 1)
    def _copy_tail():
        pltpu.sync_copy(agg_sh.at[pl.ds(_TAIL0, _TAILN)],
                        agg_out.at[cid, pl.ds(_TAIL0, _TAILN)])
        pltpu.sync_copy(deg_sh.at[pl.ds(_TAIL0, _TAILN)], dtmp_v.at[pl.ds(0, _TAILN)])

        @pl.when(cid == 0)
        def _dt0():
            pltpu.sync_copy(dtmp_v.at[pl.ds(0, _TAILN)], deg_out0.at[pl.ds(_TAIL0, _TAILN)])

        @pl.when(cid == 1)
        def _dt1():
            pltpu.sync_copy(dtmp_v.at[pl.ds(0, _TAILN)], deg_out1.at[pl.ds(_TAIL0, _TAILN)])


def _seg_sum_sc(X, src, dst):
    mesh = plsc.VectorSubcoreMesh(core_axis_name="c", subcore_axis_name="s",
                                  num_cores=_NC, num_subcores=_NS)
    z128 = jnp.zeros((_RPT, D), jnp.float32)
    fn = pl.kernel(
        _seg_sum_body,
        out_type=[jax.ShapeDtypeStruct((_NC, N_NODES, D), jnp.float32),
                  jax.ShapeDtypeStruct((N_NODES,), jnp.float32),
                  jax.ShapeDtypeStruct((N_NODES,), jnp.float32)],
        mesh=mesh,
        scratch_types=[
            pltpu.VMEM_SHARED((N_NODES, D), jnp.float32),
            pltpu.VMEM_SHARED((N_NODES,), jnp.float32),
            pltpu.VMEM((_EPW,), jnp.int32),
            pltpu.VMEM((_EPW,), jnp.int32),
            pltpu.VMEM((_CH,), jnp.int32),
            pltpu.VMEM((_CH,), jnp.int32),
            pltpu.VMEM((_CH,), jnp.int32),
            pltpu.VMEM((_CH,), jnp.int32),
            pltpu.VMEM((_CH, D), jnp.float32),
            pltpu.VMEM((_CH, D), jnp.float32),
            pltpu.VMEM((_CH,), jnp.float32),
            pltpu.VMEM((_DT,), jnp.float32),
            pltpu.SemaphoreType.DMA,
            pltpu.SemaphoreType.DMA,
            pltpu.SemaphoreType.DMA,
            pltpu.SemaphoreType.DMA,
            pltpu.SemaphoreType.DMA,
            pltpu.SemaphoreType.DMA,
        ],
    )
    return fn(X, src, dst, z128)

_RB1 = 1000
_RB2 = 1000


_RPN = 624
_TAILC = N_NET - 31 * _RPN - _RPN
_OWN = _RPN + _TAILC
_ACCR = 664
_DUM = 656
_CB = 2000
_NCB = N_CONN // _CB
_GPB = _CB // 16
_GC = 128
_CAPL = 384
_DRTH = _CAPL - _GPB
_DUMPS = _CAPL * 16
_CAPB = 6272


def _seg_max_body(h_hbm, csrc_hbm, cdst_hbm, ninf_hbm,
                  xx_out,
                  conn_sh, acc_v, srcb0, srcb1, dstb0, dstb1,
                  cpk_v, plc_v, rows_v, idx_v, sem, bs0, bs1):
    cid = lax.axis_index("c")
    sid = lax.axis_index("s")
    wid = cid * _NS + sid
    lo = wid * _RPN
    hi = lo + jnp.where(wid == _NW - 1, _OWN, _RPN)

    pltpu.sync_copy(ninf_hbm, acc_v)
    _cpt = N_CONN // _NS
    for hbm_ref, sbase in ((csrc_hbm, 0), (cdst_hbm, N_CONN)):
        for off, sz in ((0, 2000), (2000, 2000), (4000, 1000)):
            pltpu.sync_copy(hbm_ref.at[pl.ds(sid * _cpt + off, sz)],
                            srcb0.at[pl.ds(0, sz)])
            pltpu.sync_copy(srcb0.at[pl.ds(0, sz)],
                            conn_sh.at[pl.ds(sbase + sid * _cpt + off, sz)])
    dumv = jnp.full((16,), _DUM, jnp.int32)

    def fill_dum(i, c):
        cpk_v[pl.ds(i * 16, 16)] = dumv
        return c
    lax.fori_loop(0, _CAPB // 16, fill_dum, 0)
    lanes = lax.iota(jnp.int32, 16)
    plc_v[...] = jnp.zeros((16,), jnp.int32)
    plsc.subcore_barrier()

    srcb = (srcb0, srcb1)
    dstb = (dstb0, dstb1)
    bsem = (bs0, bs1)

    def fetch_block(b, p):
        boff = b * _CB
        pltpu.async_copy(conn_sh.at[pl.ds(boff, _CB)], srcb[p], bsem[p])
        pltpu.async_copy(conn_sh.at[pl.ds(N_CONN + boff, _CB)], dstb[p], bsem[p])

    fetch_block(0, 0)
    fetch_block(1, 1)

    def block(b, c, p):
        @pl.when(b < _NCB)
        def _scan():
            pltpu.make_async_copy(conn_sh.at[pl.ds(0, _CB)], srcb[p],
                                  bsem[p]).wait()
            pltpu.make_async_copy(conn_sh.at[pl.ds(0, _CB)], dstb[p],
                                  bsem[p]).wait()

            def compact(i, c2):
                d = dstb[p][pl.ds(i * 16, 16)]
                sv = srcb[p][pl.ds(i * 16, 16)]
                m = (d >= lo) & (d < hi)
                plc = plc_v[...]
                slot = jnp.where(m, plc * 16 + lanes, _DUMPS)
                plsc.store_scatter(cpk_v, [slot], sv * 1024 + (d - lo))
                plc_v[...] = plc + jnp.where(m, 1, 0)
                return c2
            lax.fori_loop(0, _GPB, compact, 0)

            @pl.when(b + 2 < _NCB)
            def _pref():
                fetch_block(b + 2, p)

        plc = plc_v[...]
        do_drain = jnp.any(plc >= _DRTH) | (b == _NCB)

        @pl.when(do_drain)
        def _drain():
            mx = plc[0]
            for e in range(1, 16):
                mx = jnp.maximum(mx, plc[e])
            nch = (mx + 7) // 8

            def chunk(rc, c4):
                @pl.when(rc < nch)
                def _chunk():
                    for g in range(_GC // 16):
                        idx_v[pl.ds(g * 16, 16)] = (
                            cpk_v[pl.ds(rc * _GC + g * 16, 16)] >> 10)
                    pltpu.async_copy(h_hbm.at[idx_v], rows_v, sem).wait()

                    def group(g, c3):
                        dloc = cpk_v[pl.ds(rc * _GC + g * 16, 16)] & 1023
                        for e in range(16):
                            dd = dloc[e]
                            r = g * 16 + e
                            for cc in range(8):
                                cs = pl.ds(cc * 16, 16)
                                acc_v[dd, cs] = jnp.maximum(acc_v[dd, cs],
                                                            rows_v[r, cs])
                        return c3
                    lax.fori_loop(0, _GC // 16, group, 0)
                return c4
            lax.fori_loop(0, _CAPL // 8, chunk, 0)
            plc_v[...] = jnp.zeros((16,), jnp.int32)
        return c

    def blockpair(i, c):
        for p in range(2):
            b = 2 * i + p

            @pl.when(b <= _NCB)
            def _b():
                block(b, 0, p)
        return c
    lax.fori_loop(0, (_NCB + 2 + 1) // 2, blockpair, 0)

    pltpu.sync_copy(acc_v.at[pl.ds(0, _RPN)], xx_out.at[pl.ds(lo, _RPN)])

    @pl.when(wid == _NW - 1)
    def _tail():
        pltpu.sync_copy(acc_v.at[pl.ds(_RPN, _TAILC)],
                        xx_out.at[pl.ds(31 * _RPN + _RPN, _TAILC)])


def _seg_max_sc(h, conn_src, conn_dst):
    mesh = plsc.VectorSubcoreMesh(core_axis_name="c", subcore_axis_name="s",
                                  num_cores=_NC, num_subcores=_NS)
    ninf = jnp.full((_ACCR, D), -jnp.inf, jnp.float32)
    fn = pl.kernel(
        _seg_max_body,
        out_type=jax.ShapeDtypeStruct((N_NET, D), jnp.float32),
        mesh=mesh,
        scratch_types=[
            pltpu.VMEM_SHARED((2 * N_CONN,), jnp.int32),
            pltpu.VMEM((_ACCR, D), jnp.float32),
            pltpu.VMEM((_CB,), jnp.int32),
            pltpu.VMEM((_CB,), jnp.int32),
            pltpu.VMEM((_CB,), jnp.int32),
            pltpu.VMEM((_CB,), jnp.int32),
            pltpu.VMEM((_CAPB,), jnp.int32),
            pltpu.VMEM((16,), jnp.int32),
            pltpu.VMEM((_GC, D), jnp.float32),
            pltpu.VMEM((_GC,), jnp.int32),
            pltpu.SemaphoreType.DMA,
            pltpu.SemaphoreType.DMA,
            pltpu.SemaphoreType.DMA,
        ],
        compiler_params=pltpu.CompilerParams(needs_layout_passes=False),
    )
    return fn(h, conn_src, conn_dst, ninf)


def _sage_dense_body(x_ref, aggp_ref, degp_ref, ws_ref, wn_ref, b_ref, o_ref):
    agg = aggp_ref[0] + aggp_ref[1]
    deg = degp_ref[:, 0] + degp_ref[:, 1]
    inv = 1.0 / jnp.clip(deg, 1.0, None)
    hn = agg * inv[:, None]
    h = (jnp.dot(x_ref[...], ws_ref[...], preferred_element_type=jnp.float32)
         + jnp.dot(hn, wn_ref[...], preferred_element_type=jnp.float32)
         + b_ref[...])
    o_ref[...] = jnp.where(h >= 0.0, h, 0.01 * h)


def _sage_dense(X, agg_parts, deg_parts, W_self, W_neigh, b_sage):
    grid = (N_NODES // _RB1,)
    return pl.pallas_call(
        _sage_dense_body,
        grid=grid,
        in_specs=[
            pl.BlockSpec((_RB1, D), lambda i: (i, 0)),
            pl.BlockSpec((2, _RB1, D), lambda i: (0, i, 0)),
            pl.BlockSpec((_RB1, 2), lambda i: (i, 0)),
            pl.BlockSpec((D, D), lambda i: (0, 0)),
            pl.BlockSpec((D, D), lambda i: (0, 0)),
            pl.BlockSpec((1, D), lambda i: (0, 0)),
        ],
        out_specs=pl.BlockSpec((_RB1, D), lambda i: (i, 0)),
        out_shape=jax.ShapeDtypeStruct((N_NODES, D), jnp.float32),
    )(X, agg_parts, deg_parts, W_self, W_neigh, b_sage.reshape(1, D))


def _mlp_body(xx_ref, w1_ref, b1_ref, w2_ref, b2_ref, o_ref):
    xx = xx_ref[...]
    xx = jnp.where(xx == -jnp.inf, 0.0, xx)
    l1 = (jnp.dot(xx, w1_ref[...], preferred_element_type=jnp.float32)
          + b1_ref[...])
    l1 = jnp.where(l1 >= 0.0, l1, 0.01 * l1)
    o_ref[...] = jnp.tanh(
        jnp.dot(l1, w2_ref[...], preferred_element_type=jnp.float32) + b2_ref[...])


def _mlp(xx, W1, b1, W2, b2):
    grid = (N_NET // _RB2,)
    return pl.pallas_call(
        _mlp_body,
        grid=grid,
        in_specs=[
            pl.BlockSpec((_RB2, D), lambda i: (i, 0)),
            pl.BlockSpec((D, H1), lambda i: (0, 0)),
            pl.BlockSpec((1, H1), lambda i: (0, 0)),
            pl.BlockSpec((H1, 1), lambda i: (0, 0)),
            pl.BlockSpec((1, 1), lambda i: (0, 0)),
        ],
        out_specs=pl.BlockSpec((_RB2, 1), lambda i: (i, 0)),
        out_shape=jax.ShapeDtypeStruct((N_NET, 1), jnp.float32),
    )(xx, W1, b1.reshape(1, H1), W2, b2.reshape(1, 1))


def kernel(X, W_self, W_neigh, b_sage, W1, b1, W2, b2, edge_index, conn_src, conn_dst):
    src = edge_index[0]
    dst = edge_index[1]
    agg_parts, deg0, deg1 = _seg_sum_sc(X, src, dst)
    deg_parts = jnp.stack([deg0, deg1], axis=1)
    h = _sage_dense(X, agg_parts, deg_parts, W_self, W_neigh, b_sage)
    xx = _seg_max_sc(h, conn_src, conn_dst)
    return _mlp(xx, W1, b1, W2, b2)
